# Initial kernel scaffold; baseline (speedup 1.0000x reference)
import jax
import jax.numpy as jnp
from jax.experimental import pallas as pl


def kernel(x, edge_index, edge_attr, u, batch, We1, be1, We2, be2, Wn1, bn1, Wn2, bn2, Wg1, bg1, Wg2, bg2):
    raise NotImplementedError("write your pallas kernel here")


# speedup vs baseline: 2.8862x; 2.8862x over previous
import functools

import jax
import jax.numpy as jnp
from jax import lax
from jax.experimental import pallas as pl
from jax.experimental.pallas import tpu as pltpu
from jax.experimental.pallas import tpu_sc as plsc

N = 10000
E = 320000
D = 128
H = 256
B = 64

NC = 2
NS = 16
NW = NC * NS

CHUNK = 128
NCHUNKS = E // CHUNK
GATHER_ITERS = -(-NCHUNKS // NW)
CORE_CHUNKS = NCHUNKS // NC
SCATTER_ITERS = -(-CORE_CHUNKS // NS)
NODE_STRIPE = 624
TAIL_BASE = NODE_STRIPE * NS

_mesh = functools.partial(
    plsc.VectorSubcoreMesh, core_axis_name="c", subcore_axis_name="s",
    num_cores=NC, num_subcores=NS)


def _gather_body(x_hbm, row_hbm, col_hbm, src_hbm, dst_hbm,
                 idx_v, rows_v, sem):
    cid = lax.axis_index("c")
    sid = lax.axis_index("s")
    wid = sid * NC + cid

    def step(t, carry):
        ch = wid + NW * t

        @pl.when(ch < NCHUNKS)
        def _():
            base = ch * CHUNK
            pltpu.sync_copy(row_hbm.at[pl.ds(base, CHUNK)], idx_v.at[0])
            pltpu.async_copy(x_hbm.at[idx_v.at[0]], rows_v, sem).wait()
            pltpu.sync_copy(rows_v, src_hbm.at[pl.ds(base, CHUNK)])
            pltpu.sync_copy(col_hbm.at[pl.ds(base, CHUNK)], idx_v.at[1])
            pltpu.async_copy(x_hbm.at[idx_v.at[1]], rows_v, sem).wait()
            pltpu.sync_copy(rows_v, dst_hbm.at[pl.ds(base, CHUNK)])

        return carry

    lax.fori_loop(0, GATHER_ITERS, step, 0)


def _sc_gather(x, row, col):
    return pl.kernel(
        _gather_body,
        out_type=(jax.ShapeDtypeStruct((E, D), jnp.float32),
                  jax.ShapeDtypeStruct((E, D), jnp.float32)),
        mesh=_mesh(),
        scratch_types=[
            pltpu.VMEM((2, CHUNK), jnp.int32),
            pltpu.VMEM((CHUNK, D), jnp.float32),
            pltpu.SemaphoreType.DMA,
        ],
    )(x, row, col)


def _scatter_body(e_hbm, col_hbm, zeros_hbm, agg_hbm,
                  idx_v, rows_v, acc_sp, sem):
    cid = lax.axis_index("c")
    sid = lax.axis_index("s")

    stripe = sid * NODE_STRIPE
    pltpu.sync_copy(zeros_hbm.at[pl.ds(stripe, NODE_STRIPE)],
                    acc_sp.at[pl.ds(stripe, NODE_STRIPE)])
    tail = TAIL_BASE + sid * 8

    @pl.when(sid < (N - TAIL_BASE) // 8)
    def _():
        pltpu.sync_copy(zeros_hbm.at[pl.ds(tail, 8)], acc_sp.at[pl.ds(tail, 8)])

    plsc.subcore_barrier()

    def step(t, carry):
        k = sid + NS * t

        @pl.when(k < CORE_CHUNKS)
        def _():
            base = (cid * CORE_CHUNKS + k) * CHUNK
            pltpu.sync_copy(col_hbm.at[pl.ds(base, CHUNK)], idx_v.at[0])
            pltpu.sync_copy(e_hbm.at[pl.ds(base, CHUNK)], rows_v)
            pltpu.sync_copy(rows_v, acc_sp.at[idx_v.at[0]], add=True)

        return carry

    lax.fori_loop(0, SCATTER_ITERS, step, 0)
    plsc.subcore_barrier()
    pltpu.sync_copy(acc_sp.at[pl.ds(stripe, NODE_STRIPE)],
                    agg_hbm.at[cid, pl.ds(stripe, NODE_STRIPE)])

    @pl.when(sid < (N - TAIL_BASE) // 8)
    def _():
        pltpu.sync_copy(acc_sp.at[pl.ds(tail, 8)],
                        agg_hbm.at[cid, pl.ds(tail, 8)])


def _sc_scatter(e_new, col, zeros):
    return pl.kernel(
        _scatter_body,
        out_type=jax.ShapeDtypeStruct((NC, N, D), jnp.float32),
        mesh=_mesh(),
        scratch_types=[
            pltpu.VMEM((1, CHUNK), jnp.int32),
            pltpu.VMEM((CHUNK, D), jnp.float32),
            pltpu.VMEM_SHARED((N, D), jnp.float32),
            pltpu.SemaphoreType.DMA,
        ],
    )(e_new, col, zeros)


EB = 1280


def _edge_mlp_body(src, dst, ea, w1a, w1b, w1c, b1, w2, b2, out):
    h = jnp.dot(src[...], w1a[...], preferred_element_type=jnp.float32)
    h += jnp.dot(dst[...], w1b[...], preferred_element_type=jnp.float32)
    h += jnp.dot(ea[...], w1c[...], preferred_element_type=jnp.float32)
    h = jnp.maximum(h + b1[...], 0.0)
    out[...] = jnp.dot(h, w2[...], preferred_element_type=jnp.float32) + b2[...]


def _tc_edge_mlp(src, dst, ea, w1a, w1b, w1c, b1, w2, b2):
    rows = lambda i: (i, 0)
    full = lambda i: (0, 0)
    return pl.pallas_call(
        _edge_mlp_body,
        grid=(E // EB,),
        in_specs=[
            pl.BlockSpec((EB, D), rows),
            pl.BlockSpec((EB, D), rows),
            pl.BlockSpec((EB, D), rows),
            pl.BlockSpec((D, H), full),
            pl.BlockSpec((D, H), full),
            pl.BlockSpec((D, H), full),
            pl.BlockSpec((1, H), full),
            pl.BlockSpec((H, D), full),
            pl.BlockSpec((1, D), full),
        ],
        out_specs=pl.BlockSpec((EB, D), rows),
        out_shape=jax.ShapeDtypeStruct((E, D), jnp.float32),
    )(src, dst, ea, w1a, w1b, w1c, b1, w2, b2)


NB = 1000
NGRID = N // NB


def _node_global_body(x, agg, e1, batch, wn1a, wn1b, bn1, wn2, bn2,
                      wg1a, wg1b, bg1, wg2, bg2,
                      x_out, u_out, nsum, esum, cnt):
    i = pl.program_id(0)

    @pl.when(i == 0)
    def _():
        nsum[...] = jnp.zeros_like(nsum)
        esum[...] = jnp.zeros_like(esum)
        cnt[...] = jnp.zeros_like(cnt)

    a = agg[0] + agg[1]
    h = jnp.dot(x[...], wn1a[...], preferred_element_type=jnp.float32)
    h += jnp.dot(a, wn1b[...], preferred_element_type=jnp.float32)
    h = jnp.maximum(h + bn1[...], 0.0)
    xn = jnp.dot(h, wn2[...], preferred_element_type=jnp.float32) + bn2[...]
    x_out[...] = xn

    gid = lax.broadcasted_iota(jnp.int32, (B, NB), 0)
    oh = (gid == batch[0]).astype(jnp.float32)
    nsum[...] += jnp.dot(oh, xn, preferred_element_type=jnp.float32)
    esum[...] += jnp.dot(oh, e1[...], preferred_element_type=jnp.float32)
    cnt[...] += jnp.broadcast_to(jnp.sum(oh, axis=1, keepdims=True), (B, D))

    @pl.when(i == NGRID - 1)
    def _():
        denom = cnt[...] + 1e-6
        nmean = nsum[...] / denom
        emean = esum[...] / denom
        hg = jnp.dot(nmean, wg1a[...], preferred_element_type=jnp.float32)
        hg += jnp.dot(emean, wg1b[...], preferred_element_type=jnp.float32)
        hg = jnp.maximum(hg + bg1[...], 0.0)
        u_out[...] = jnp.dot(hg, wg2[...],
                             preferred_element_type=jnp.float32) + bg2[...]


def _tc_node_global(x, agg, e_new, batch3, wn1a, wn1b, bn1, wn2, bn2,
                    wg1a, wg1b, bg1, wg2, bg2):
    rows = lambda i: (i, 0)
    full = lambda i: (0, 0)
    return pl.pallas_call(
        _node_global_body,
        grid=(NGRID,),
        in_specs=[
            pl.BlockSpec((NB, D), rows),
            pl.BlockSpec((NC, NB, D), lambda i: (0, i, 0)),
            pl.BlockSpec((NB, D), rows),
            pl.BlockSpec((1, 1, NB), lambda i: (i, 0, 0)),
            pl.BlockSpec((D, H), full),
            pl.BlockSpec((D, H), full),
            pl.BlockSpec((1, H), full),
            pl.BlockSpec((H, D), full),
            pl.BlockSpec((1, D), full),
            pl.BlockSpec((D, H), full),
            pl.BlockSpec((D, H), full),
            pl.BlockSpec((1, H), full),
            pl.BlockSpec((H, D), full),
            pl.BlockSpec((1, D), full),
        ],
        out_specs=[
            pl.BlockSpec((NB, D), rows),
            pl.BlockSpec((B, D), full),
        ],
        out_shape=[
            jax.ShapeDtypeStruct((N, D), jnp.float32),
            jax.ShapeDtypeStruct((B, D), jnp.float32),
        ],
        scratch_shapes=[
            pltpu.VMEM((B, D), jnp.float32),
            pltpu.VMEM((B, D), jnp.float32),
            pltpu.VMEM((B, D), jnp.float32),
        ],
    )(x, agg, e_new, batch3, wn1a, wn1b, bn1, wn2, bn2,
      wg1a, wg1b, bg1, wg2, bg2)


def kernel(x, edge_index, edge_attr, u, batch,
           We1, be1, We2, be2,
           Wn1, bn1, Wn2, bn2,
           Wg1, bg1, Wg2, bg2):
    row = edge_index[0]
    col = edge_index[1]

    src, dst = _sc_gather(x, row, col)

    e_new = _tc_edge_mlp(
        src, dst, edge_attr,
        We1[:D], We1[D:2 * D], We1[2 * D:], be1.reshape(1, H),
        We2, be2.reshape(1, D))

    agg = _sc_scatter(e_new, col, jnp.zeros((N, D), jnp.float32))

    batch3 = batch.reshape(NGRID, 1, NB)
    x_new, u_new = _tc_node_global(
        x, agg, e_new, batch3,
        Wn1[:D], Wn1[D:], bn1.reshape(1, H), Wn2, bn2.reshape(1, D),
        Wg1[:D], Wg1[D:], bg1.reshape(1, H), Wg2, bg2.reshape(1, D))

    return x_new, e_new, u_new


# speedup vs baseline: 2.9645x; 1.0271x over previous
import functools

import jax
import jax.numpy as jnp
from jax import lax
from jax.experimental import pallas as pl
from jax.experimental.pallas import tpu as pltpu
from jax.experimental.pallas import tpu_sc as plsc

N = 10000
E = 320000
D = 128
H = 256
B = 64

NC = 2
NS = 16
NW = NC * NS

PIECES = 4
EP = E // PIECES
CHUNK = 128
PCHUNKS = EP // CHUNK
ITERS = -(-PCHUNKS // NW)
NODE_STRIPE = 624
TAIL_BASE = NODE_STRIPE * NS

_mesh = functools.partial(
    plsc.VectorSubcoreMesh, core_axis_name="c", subcore_axis_name="s",
    num_cores=NC, num_subcores=NS)


def _gather_body(x_hbm, row_hbm, col_hbm, src_hbm, dst_hbm,
                 idx_v, rows_v, sem):
    cid = lax.axis_index("c")
    sid = lax.axis_index("s")
    wid = sid * NC + cid

    def step(t, carry):
        ch = wid + NW * t

        @pl.when(ch < PCHUNKS)
        def _():
            base = ch * CHUNK
            pltpu.sync_copy(row_hbm.at[pl.ds(base, CHUNK)], idx_v.at[0])
            pltpu.async_copy(x_hbm.at[idx_v.at[0]], rows_v, sem).wait()
            pltpu.sync_copy(rows_v, src_hbm.at[pl.ds(base, CHUNK)])
            pltpu.sync_copy(col_hbm.at[pl.ds(base, CHUNK)], idx_v.at[1])
            pltpu.async_copy(x_hbm.at[idx_v.at[1]], rows_v, sem).wait()
            pltpu.sync_copy(rows_v, dst_hbm.at[pl.ds(base, CHUNK)])

        return carry

    lax.fori_loop(0, ITERS, step, 0)


def _sc_gather(x, row, col):
    return pl.kernel(
        _gather_body,
        out_type=(jax.ShapeDtypeStruct((EP, D), jnp.float32),
                  jax.ShapeDtypeStruct((EP, D), jnp.float32)),
        mesh=_mesh(),
        scratch_types=[
            pltpu.VMEM((2, CHUNK), jnp.int32),
            pltpu.VMEM((CHUNK, D), jnp.float32),
            pltpu.SemaphoreType.DMA,
        ],
    )(x, row, col)


def _scatter_body(e_hbm, col_hbm, zeros_hbm, agg_hbm,
                  idx_v, rows_v, acc_sp, sem):
    cid = lax.axis_index("c")
    sid = lax.axis_index("s")
    wid = sid * NC + cid

    stripe = sid * NODE_STRIPE
    pltpu.sync_copy(zeros_hbm.at[pl.ds(stripe, NODE_STRIPE)],
                    acc_sp.at[pl.ds(stripe, NODE_STRIPE)])
    tail = TAIL_BASE + sid * 8

    @pl.when(sid < (N - TAIL_BASE) // 8)
    def _():
        pltpu.sync_copy(zeros_hbm.at[pl.ds(tail, 8)], acc_sp.at[pl.ds(tail, 8)])

    plsc.subcore_barrier()

    def step(t, carry):
        ch = wid + NW * t

        @pl.when(ch < PCHUNKS)
        def _():
            base = ch * CHUNK
            pltpu.sync_copy(col_hbm.at[pl.ds(base, CHUNK)], idx_v.at[0])
            pltpu.sync_copy(e_hbm.at[pl.ds(base, CHUNK)], rows_v)
            pltpu.sync_copy(rows_v, acc_sp.at[idx_v.at[0]], add=True)

        return carry

    lax.fori_loop(0, ITERS, step, 0)
    plsc.subcore_barrier()
    pltpu.sync_copy(acc_sp.at[pl.ds(stripe, NODE_STRIPE)],
                    agg_hbm.at[cid, pl.ds(stripe, NODE_STRIPE)])

    @pl.when(sid < (N - TAIL_BASE) // 8)
    def _():
        pltpu.sync_copy(acc_sp.at[pl.ds(tail, 8)],
                        agg_hbm.at[cid, pl.ds(tail, 8)])


def _sc_scatter(e_new, col, zeros):
    return pl.kernel(
        _scatter_body,
        out_type=jax.ShapeDtypeStruct((NC, N, D), jnp.float32),
        mesh=_mesh(),
        scratch_types=[
            pltpu.VMEM((1, CHUNK), jnp.int32),
            pltpu.VMEM((CHUNK, D), jnp.float32),
            pltpu.VMEM_SHARED((N, D), jnp.float32),
            pltpu.SemaphoreType.DMA,
        ],
    )(e_new, col, zeros)


EB = 1600


def _edge_mlp_math(src, dst, ea, w1a, w1b, w1c, b1, w2, b2):
    h = jnp.dot(src[...], w1a[...], preferred_element_type=jnp.float32)
    h += jnp.dot(dst[...], w1b[...], preferred_element_type=jnp.float32)
    h += jnp.dot(ea[...], w1c[...], preferred_element_type=jnp.float32)
    h = jnp.maximum(h + b1[...], 0.0)
    return jnp.dot(h, w2[...], preferred_element_type=jnp.float32) + b2[...]


def _edge_mlp_body0(src, dst, ea, w1a, w1b, w1c, b1, w2, b2, out, full_out):
    e = _edge_mlp_math(src, dst, ea, w1a, w1b, w1c, b1, w2, b2)
    out[...] = e
    full_out[...] = e


def _edge_mlp_bodyk(src, dst, ea, w1a, w1b, w1c, b1, w2, b2, buf,
                    out, full_out):
    del buf
    e = _edge_mlp_math(src, dst, ea, w1a, w1b, w1c, b1, w2, b2)
    out[...] = e
    full_out[...] = e


def _tc_edge_mlp(k, src, dst, ea, w1a, w1b, w1c, b1, w2, b2, e_buf):
    rows = lambda i: (i, 0)
    off = k * (EP // EB)
    rows_off = lambda i: (i + off, 0)
    full = lambda i: (0, 0)
    in_specs = [
        pl.BlockSpec((EB, D), rows),
        pl.BlockSpec((EB, D), rows),
        pl.BlockSpec((EB, D), rows),
        pl.BlockSpec((D, H), full),
        pl.BlockSpec((D, H), full),
        pl.BlockSpec((D, H), full),
        pl.BlockSpec((1, H), full),
        pl.BlockSpec((H, D), full),
        pl.BlockSpec((1, D), full),
    ]
    args = (src, dst, ea, w1a, w1b, w1c, b1, w2, b2)
    if k == 0:
        body = _edge_mlp_body0
        aliases = {}
    else:
        body = _edge_mlp_bodyk
        in_specs.append(pl.BlockSpec(memory_space=pl.ANY))
        args = args + (e_buf,)
        aliases = {9: 1}
    return pl.pallas_call(
        body,
        grid=(EP // EB,),
        in_specs=in_specs,
        out_specs=[pl.BlockSpec((EB, D), rows),
                   pl.BlockSpec((EB, D), rows_off)],
        out_shape=[jax.ShapeDtypeStruct((EP, D), jnp.float32),
                   jax.ShapeDtypeStruct((E, D), jnp.float32)],
        input_output_aliases=aliases,
    )(*args)


NB = 1000
NGRID = N // NB


def _node_global_body(x, a0, a1, a2, a3, e1, batch, wn1a, wn1b, bn1, wn2, bn2,
                      wg1a, wg1b, bg1, wg2, bg2,
                      x_out, u_out, nsum, esum, cnt):
    i = pl.program_id(0)

    @pl.when(i == 0)
    def _():
        nsum[...] = jnp.zeros_like(nsum)
        esum[...] = jnp.zeros_like(esum)
        cnt[...] = jnp.zeros_like(cnt)

    a = a0[0] + a0[1] + a1[0] + a1[1] + a2[0] + a2[1] + a3[0] + a3[1]
    h = jnp.dot(x[...], wn1a[...], preferred_element_type=jnp.float32)
    h += jnp.dot(a, wn1b[...], preferred_element_type=jnp.float32)
    h = jnp.maximum(h + bn1[...], 0.0)
    xn = jnp.dot(h, wn2[...], preferred_element_type=jnp.float32) + bn2[...]
    x_out[...] = xn

    gid = lax.broadcasted_iota(jnp.int32, (B, NB), 0)
    oh = (gid == batch[0]).astype(jnp.float32)
    nsum[...] += jnp.dot(oh, xn, preferred_element_type=jnp.float32)
    esum[...] += jnp.dot(oh, e1[...], preferred_element_type=jnp.float32)
    cnt[...] += jnp.broadcast_to(jnp.sum(oh, axis=1, keepdims=True), (B, D))

    @pl.when(i == NGRID - 1)
    def _():
        denom = cnt[...] + 1e-6
        nmean = nsum[...] / denom
        emean = esum[...] / denom
        hg = jnp.dot(nmean, wg1a[...], preferred_element_type=jnp.float32)
        hg += jnp.dot(emean, wg1b[...], preferred_element_type=jnp.float32)
        hg = jnp.maximum(hg + bg1[...], 0.0)
        u_out[...] = jnp.dot(hg, wg2[...],
                             preferred_element_type=jnp.float32) + bg2[...]


def _tc_node_global(x, aggs, e_new0, batch3, wn1a, wn1b, bn1, wn2, bn2,
                    wg1a, wg1b, bg1, wg2, bg2):
    rows = lambda i: (i, 0)
    full = lambda i: (0, 0)
    agg_spec = pl.BlockSpec((NC, NB, D), lambda i: (0, i, 0))
    return pl.pallas_call(
        _node_global_body,
        grid=(NGRID,),
        in_specs=[
            pl.BlockSpec((NB, D), rows),
            agg_spec, agg_spec, agg_spec, agg_spec,
            pl.BlockSpec((NB, D), rows),
            pl.BlockSpec((1, 1, NB), lambda i: (i, 0, 0)),
            pl.BlockSpec((D, H), full),
            pl.BlockSpec((D, H), full),
            pl.BlockSpec((1, H), full),
            pl.BlockSpec((H, D), full),
            pl.BlockSpec((1, D), full),
            pl.BlockSpec((D, H), full),
            pl.BlockSpec((D, H), full),
            pl.BlockSpec((1, H), full),
            pl.BlockSpec((H, D), full),
            pl.BlockSpec((1, D), full),
        ],
        out_specs=[
            pl.BlockSpec((NB, D), rows),
            pl.BlockSpec((B, D), full),
        ],
        out_shape=[
            jax.ShapeDtypeStruct((N, D), jnp.float32),
            jax.ShapeDtypeStruct((B, D), jnp.float32),
        ],
        scratch_shapes=[
            pltpu.VMEM((B, D), jnp.float32),
            pltpu.VMEM((B, D), jnp.float32),
            pltpu.VMEM((B, D), jnp.float32),
        ],
    )(x, *aggs, e_new0, batch3, wn1a, wn1b, bn1, wn2, bn2,
      wg1a, wg1b, bg1, wg2, bg2)


def kernel(x, edge_index, edge_attr, u, batch,
           We1, be1, We2, be2,
           Wn1, bn1, Wn2, bn2,
           Wg1, bg1, Wg2, bg2):
    row = edge_index[0]
    col = edge_index[1]
    zeros = jnp.zeros((N, D), jnp.float32)

    w1a, w1b, w1c = We1[:D], We1[D:2 * D], We1[2 * D:]
    b1 = be1.reshape(1, H)
    b2 = be2.reshape(1, D)

    e_pieces = []
    aggs = []
    e_new = None
    for k in range(PIECES):
        sl = slice(k * EP, (k + 1) * EP)
        src, dst = _sc_gather(x, row[sl], col[sl])
        e_k, e_new = _tc_edge_mlp(k, src, dst, edge_attr[sl],
                                  w1a, w1b, w1c, b1, We2, b2, e_new)
        e_pieces.append(e_k)
        aggs.append(_sc_scatter(e_k, col[sl], zeros))

    batch3 = batch.reshape(NGRID, 1, NB)
    x_new, u_new = _tc_node_global(
        x, aggs, e_pieces[0], batch3,
        Wn1[:D], Wn1[D:], bn1.reshape(1, H), Wn2, bn2.reshape(1, D),
        Wg1[:D], Wg1[D:], bg1.reshape(1, H), Wg2, bg2.reshape(1, D))

    return x_new, e_new, u_new
